# initial kernel scaffold (unmeasured)
import jax
import jax.numpy as jnp
from jax import lax
from jax.experimental import pallas as pl
from jax.experimental.pallas import tpu as pltpu

N_DEV = 16

RING = [0, 1, 2, 6, 5, 4, 8, 9, 10, 14, 13, 12, 15, 11, 7, 3]
POS = [0] * N_DEV
for _r, _l in enumerate(RING):
    POS[_l] = _r


def kernel(x, w_mat):
    m_per, k = x.shape
    _, n_per = w_mat.shape

    def body(x_ref, w_ref, out_ref, comm_ref, send_sems, recv_sems,
             amax_src, amax_buf, amax_send_sems, amax_recv_sems):
        my = lax.axis_index("i")
        ring = jnp.array(RING, dtype=jnp.int32)
        pos = jnp.array(POS, dtype=jnp.int32)
        my_r = pos[my]
        right = ring[(my_r + 1) % N_DEV]
        left = ring[(my_r - 1 + N_DEV) % N_DEV]

        barrier_sem = pltpu.get_barrier_semaphore()
        for nbr in [left, right]:
            pl.semaphore_signal(
                barrier_sem, inc=1,
                device_id=(nbr,), device_id_type=pl.DeviceIdType.MESH,
            )
        pl.semaphore_wait(barrier_sem, 2)

        def mm(a):
            return lax.dot_general(
                a, w_ref[...], (((1,), (0,)), ((), ())),
                precision=lax.Precision.HIGHEST,
                preferred_element_type=jnp.float32,
            )

        comm_ref[0] = x_ref[...]

        ymax = jnp.float32(0.0)
        for h in range(N_DEV - 1):
            send_slot = h % 2
            recv_slot = (h + 1) % 2
            rdma = pltpu.make_async_remote_copy(
                src_ref=comm_ref.at[send_slot],
                dst_ref=comm_ref.at[recv_slot],
                send_sem=send_sems.at[h],
                recv_sem=recv_sems.at[h],
                device_id=(right,),
                device_id_type=pl.DeviceIdType.MESH,
            )
            rdma.start()
            origin = ring[(my_r - h + N_DEV) % N_DEV]
            y = mm(comm_ref[send_slot])
            ymax = jnp.maximum(ymax, jnp.max(y))
            out_ref[pl.ds(origin * m_per, m_per), :] = y
            rdma.wait()
        origin = ring[(my_r + 1) % N_DEV]
        y = mm(comm_ref[(N_DEV - 1) % 2])
        ymax = jnp.maximum(ymax, jnp.max(y))
        out_ref[pl.ds(origin * m_per, m_per), :] = y

        amax_src[...] = jnp.full((8, 128), ymax, dtype=jnp.float32)
        rdmas = []
        for o in range(1, N_DEV):
            tgt = ring[(my_r + o) % N_DEV]
            r = pltpu.make_async_remote_copy(
                src_ref=amax_src,
                dst_ref=amax_buf.at[o - 1],
                send_sem=amax_send_sems.at[o - 1],
                recv_sem=amax_recv_sems.at[o - 1],
                device_id=(tgt,),
                device_id_type=pl.DeviceIdType.MESH,
            )
            r.start()
            rdmas.append(r)
        for r in rdmas:
            r.wait_send()
        for r in rdmas:
            r.wait_recv()

        gmax = ymax
        for o in range(1, N_DEV):
            gmax = jnp.maximum(gmax, amax_buf[o - 1, 0, 0])

        scale = gmax / jnp.float32(127.0)
        y_all = jnp.maximum(out_ref[...], 0.0)
        q = jnp.clip(jnp.round(y_all / scale), -127.0, 127.0)
        out_ref[...] = q * scale

    return pl.pallas_call(
        body,
        out_shape=jax.ShapeDtypeStruct((N_DEV * m_per, n_per), jnp.float32),
        in_specs=[
            pl.BlockSpec(memory_space=pltpu.VMEM),
            pl.BlockSpec(memory_space=pltpu.VMEM),
        ],
        out_specs=pl.BlockSpec(memory_space=pltpu.VMEM),
        scratch_shapes=[
            pltpu.VMEM((2, m_per, k), jnp.float32),
            pltpu.SemaphoreType.DMA((N_DEV - 1,)),
            pltpu.SemaphoreType.DMA((N_DEV - 1,)),
            pltpu.VMEM((8, 128), jnp.float32),
            pltpu.VMEM((N_DEV - 1, 8, 128), jnp.float32),
            pltpu.SemaphoreType.DMA((N_DEV - 1,)),
            pltpu.SemaphoreType.DMA((N_DEV - 1,)),
        ],
        compiler_params=pltpu.CompilerParams(collective_id=0),
    )(x, w_mat)


# baseline (device time: 723362 ns/iter reference)
import jax
import jax.numpy as jnp
from jax import lax
from jax.experimental import pallas as pl
from jax.experimental.pallas import tpu as pltpu

N_DEV = 16

RING = [0, 1, 2, 6, 5, 4, 8, 9, 10, 14, 13, 12, 15, 11, 7, 3]
POS = [0] * N_DEV
for _r, _l in enumerate(RING):
    POS[_l] = _r


def kernel(x, w_mat):
    m_per, k = x.shape
    _, n_per = w_mat.shape

    def body(ring_ref, pos_ref, x_ref, w_ref, out_ref, comm_ref,
             send_sems, recv_sems,
             amax_src, amax_buf, amax_send_sems, amax_recv_sems):
        my = lax.axis_index("i")

        def ring(i):
            return ring_ref[i]

        my_r = pos_ref[my]
        right = ring((my_r + 1) % N_DEV)
        left = ring((my_r - 1 + N_DEV) % N_DEV)

        barrier_sem = pltpu.get_barrier_semaphore()
        for nbr in [left, right]:
            pl.semaphore_signal(
                barrier_sem, inc=1,
                device_id=(nbr,), device_id_type=pl.DeviceIdType.MESH,
            )
        pl.semaphore_wait(barrier_sem, 2)

        def mm(a):
            return lax.dot_general(
                a, w_ref[...], (((1,), (0,)), ((), ())),
                precision=lax.Precision.HIGHEST,
                preferred_element_type=jnp.float32,
            )

        comm_ref[0] = x_ref[...]

        ymax = jnp.float32(0.0)
        for h in range(N_DEV - 1):
            send_slot = h % 2
            recv_slot = (h + 1) % 2
            rdma = pltpu.make_async_remote_copy(
                src_ref=comm_ref.at[send_slot],
                dst_ref=comm_ref.at[recv_slot],
                send_sem=send_sems.at[h],
                recv_sem=recv_sems.at[h],
                device_id=(right,),
                device_id_type=pl.DeviceIdType.MESH,
            )
            rdma.start()
            origin = ring((my_r - h + N_DEV) % N_DEV)
            y = mm(comm_ref[send_slot])
            ymax = jnp.maximum(ymax, jnp.max(y))
            out_ref[pl.ds(origin * m_per, m_per), :] = y
            rdma.wait()
        origin = ring((my_r + 1) % N_DEV)
        y = mm(comm_ref[(N_DEV - 1) % 2])
        ymax = jnp.maximum(ymax, jnp.max(y))
        out_ref[pl.ds(origin * m_per, m_per), :] = y

        amax_src[...] = jnp.full((8, 128), ymax, dtype=jnp.float32)
        rdmas = []
        for o in range(1, N_DEV):
            tgt = ring((my_r + o) % N_DEV)
            r = pltpu.make_async_remote_copy(
                src_ref=amax_src,
                dst_ref=amax_buf.at[o - 1],
                send_sem=amax_send_sems.at[o - 1],
                recv_sem=amax_recv_sems.at[o - 1],
                device_id=(tgt,),
                device_id_type=pl.DeviceIdType.MESH,
            )
            r.start()
            rdmas.append(r)
        for r in rdmas:
            r.wait_send()
        for r in rdmas:
            r.wait_recv()

        gmax = ymax
        for o in range(1, N_DEV):
            gmax = jnp.maximum(gmax, amax_buf[o - 1, 0, 0])

        scale = gmax / jnp.float32(127.0)
        y_all = jnp.maximum(out_ref[...], 0.0)
        q = jnp.clip(jnp.round(y_all / scale), -127.0, 127.0)
        out_ref[...] = q * scale

    return pl.pallas_call(
        body,
        out_shape=jax.ShapeDtypeStruct((N_DEV * m_per, n_per), jnp.float32),
        in_specs=[
            pl.BlockSpec(memory_space=pltpu.SMEM),
            pl.BlockSpec(memory_space=pltpu.SMEM),
            pl.BlockSpec(memory_space=pltpu.VMEM),
            pl.BlockSpec(memory_space=pltpu.VMEM),
        ],
        out_specs=pl.BlockSpec(memory_space=pltpu.VMEM),
        scratch_shapes=[
            pltpu.VMEM((2, m_per, k), jnp.float32),
            pltpu.SemaphoreType.DMA((N_DEV - 1,)),
            pltpu.SemaphoreType.DMA((N_DEV - 1,)),
            pltpu.VMEM((8, 128), jnp.float32),
            pltpu.VMEM((N_DEV - 1, 8, 128), jnp.float32),
            pltpu.SemaphoreType.DMA((N_DEV - 1,)),
            pltpu.SemaphoreType.DMA((N_DEV - 1,)),
        ],
        compiler_params=pltpu.CompilerParams(collective_id=0),
    )(jnp.array(RING, dtype=jnp.int32), jnp.array(POS, dtype=jnp.int32), x, w_mat)


# device time: 233054 ns/iter; 3.1038x vs baseline; 3.1038x over previous
import jax
import jax.numpy as jnp
from jax import lax
from jax.experimental import pallas as pl
from jax.experimental.pallas import tpu as pltpu

N_DEV = 16
CW = 8
CCW = 7

RING = [0, 1, 2, 6, 5, 4, 8, 9, 10, 14, 13, 12, 15, 11, 7, 3]
POS = [0] * N_DEV
for _r, _l in enumerate(RING):
    POS[_l] = _r


def kernel(x, w_mat):
    m_per, k = x.shape
    _, n_per = w_mat.shape

    def body(ring_ref, pos_ref, x_ref, w_ref, out_ref,
             wfull, tile_buf,
             cw_send, cw_recv, ccw_send, ccw_recv,
             a2a_send_sems, a2a_recv_sems,
             amax_src, amax_buf, amax_send_sems, amax_recv_sems):
        my = lax.axis_index("i")

        def ring(i):
            return ring_ref[i]

        my_r = pos_ref[my]
        right = ring((my_r + 1) % N_DEV)
        left = ring((my_r - 1 + N_DEV) % N_DEV)

        barrier_sem = pltpu.get_barrier_semaphore()
        for nbr in [left, right]:
            pl.semaphore_signal(
                barrier_sem, inc=1,
                device_id=(nbr,), device_id_type=pl.DeviceIdType.MESH,
            )
        pl.semaphore_wait(barrier_sem, 2)

        def mm(a):
            return lax.dot_general(
                x_ref[...], a, (((1,), (0,)), ((), ())),
                precision=lax.Precision.HIGHEST,
                preferred_element_type=jnp.float32,
            )

        wfull[pl.ds(my_r, 1)] = w_ref[...][None]

        ymax = jnp.float32(0.0)
        a2a_rdmas = []

        def send_tile(p, s_idx):
            nonlocal ymax
            tgt = ring(p)
            y = mm(wfull[p])
            ymax = jnp.maximum(ymax, jnp.max(y))
            tile_buf[pl.ds(p, 1)] = y[None]
            r = pltpu.make_async_remote_copy(
                src_ref=tile_buf.at[p],
                dst_ref=out_ref.at[pl.ds(my * m_per, m_per), :],
                send_sem=a2a_send_sems.at[s_idx],
                recv_sem=a2a_recv_sems.at[s_idx],
                device_id=(tgt,),
                device_id_type=pl.DeviceIdType.MESH,
            )
            r.start()
            a2a_rdmas.append(r)

        for s in range(CW + 1):
            waits = []
            if s < CW:
                slot = (my_r - s + N_DEV) % N_DEV
                r1 = pltpu.make_async_remote_copy(
                    src_ref=wfull.at[slot],
                    dst_ref=wfull.at[slot],
                    send_sem=cw_send.at[s],
                    recv_sem=cw_recv.at[s],
                    device_id=(right,),
                    device_id_type=pl.DeviceIdType.MESH,
                )
                r1.start()
                waits.append(r1)
            if s < CCW:
                slot2 = (my_r + s) % N_DEV
                r2 = pltpu.make_async_remote_copy(
                    src_ref=wfull.at[slot2],
                    dst_ref=wfull.at[slot2],
                    send_sem=ccw_send.at[s],
                    recv_sem=ccw_recv.at[s],
                    device_id=(left,),
                    device_id_type=pl.DeviceIdType.MESH,
                )
                r2.start()
                waits.append(r2)
            if s == 0:
                y0 = mm(w_ref[...])
                ymax = jnp.maximum(ymax, jnp.max(y0))
                out_ref[pl.ds(my * m_per, m_per), :] = y0
            else:
                send_tile((my_r - s + N_DEV) % N_DEV, 15 - s)
                if s <= CCW:
                    send_tile((my_r + s) % N_DEV, s - 1)
            for r in waits:
                r.wait()

        for r in a2a_rdmas:
            r.wait_send()

        amax_src[...] = jnp.full((8, 128), ymax, dtype=jnp.float32)
        amax_rdmas = []
        for o in range(1, N_DEV):
            tgt = ring((my_r + o) % N_DEV)
            r = pltpu.make_async_remote_copy(
                src_ref=amax_src,
                dst_ref=amax_buf.at[o - 1],
                send_sem=amax_send_sems.at[o - 1],
                recv_sem=amax_recv_sems.at[o - 1],
                device_id=(tgt,),
                device_id_type=pl.DeviceIdType.MESH,
            )
            r.start()
            amax_rdmas.append(r)
        for r in amax_rdmas:
            r.wait_send()
        for r in amax_rdmas:
            r.wait_recv()
        gmax = ymax
        for o in range(1, N_DEV):
            gmax = jnp.maximum(gmax, amax_buf[o - 1, 0, 0])

        for r in a2a_rdmas:
            r.wait_recv()

        scale = gmax / jnp.float32(127.0)
        y_all = jnp.maximum(out_ref[...], 0.0)
        q = jnp.clip(jnp.round(y_all / scale), -127.0, 127.0)
        out_ref[...] = q * scale

    return pl.pallas_call(
        body,
        out_shape=jax.ShapeDtypeStruct((N_DEV * m_per, n_per), jnp.float32),
        in_specs=[
            pl.BlockSpec(memory_space=pltpu.SMEM),
            pl.BlockSpec(memory_space=pltpu.SMEM),
            pl.BlockSpec(memory_space=pltpu.VMEM),
            pl.BlockSpec(memory_space=pltpu.VMEM),
        ],
        out_specs=pl.BlockSpec(memory_space=pltpu.VMEM),
        scratch_shapes=[
            pltpu.VMEM((N_DEV, k, n_per), jnp.float32),
            pltpu.VMEM((N_DEV, m_per, n_per), jnp.float32),
            pltpu.SemaphoreType.DMA((CW,)),
            pltpu.SemaphoreType.DMA((CW,)),
            pltpu.SemaphoreType.DMA((CCW,)),
            pltpu.SemaphoreType.DMA((CCW,)),
            pltpu.SemaphoreType.DMA((N_DEV - 1,)),
            pltpu.SemaphoreType.DMA((N_DEV - 1,)),
            pltpu.VMEM((8, 128), jnp.float32),
            pltpu.VMEM((N_DEV - 1, 8, 128), jnp.float32),
            pltpu.SemaphoreType.DMA((N_DEV - 1,)),
            pltpu.SemaphoreType.DMA((N_DEV - 1,)),
        ],
        compiler_params=pltpu.CompilerParams(
            collective_id=0, vmem_limit_bytes=56 * 1024 * 1024,
        ),
    )(jnp.array(RING, dtype=jnp.int32), jnp.array(POS, dtype=jnp.int32),
      x, w_mat)


# device time: 219865 ns/iter; 3.2900x vs baseline; 1.0600x over previous
import jax
import jax.numpy as jnp
from jax import lax
from jax.experimental import pallas as pl
from jax.experimental.pallas import tpu as pltpu

N_DEV = 16
CW = 8
CCW = 8

RING = [0, 1, 2, 6, 5, 4, 8, 9, 10, 14, 13, 12, 15, 11, 7, 3]
POS = [0] * N_DEV
for _r, _l in enumerate(RING):
    POS[_l] = _r


def kernel(x, w_mat):
    m_per, k = x.shape
    _, n_per = w_mat.shape

    def body(ring_ref, pos_ref, x_ref, w_ref, out_ref,
             wfull, tile_buf,
             cw_send, cw_recv, ccw_send, ccw_recv,
             a2a_send_sems, a2a_recv_sems,
             amax_src, amax_buf, amax_send_sems, amax_recv_sems):
        my = lax.axis_index("i")

        def ring(i):
            return ring_ref[i]

        my_r = pos_ref[my]
        right = ring((my_r + 1) % N_DEV)
        left = ring((my_r - 1 + N_DEV) % N_DEV)

        barrier_sem = pltpu.get_barrier_semaphore()
        for nbr in [left, right]:
            pl.semaphore_signal(
                barrier_sem, inc=1,
                device_id=(nbr,), device_id_type=pl.DeviceIdType.MESH,
            )
        pl.semaphore_wait(barrier_sem, 2)

        def mm(a):
            return lax.dot_general(
                x_ref[...], a, (((1,), (0,)), ((), ())),
                precision=lax.Precision.HIGHEST,
                preferred_element_type=jnp.float32,
            )

        wfull[pl.ds(my_r, 1)] = w_ref[...][None]

        ymax = jnp.float32(0.0)
        a2a_rdmas = []

        def send_tile(p, s_idx):
            nonlocal ymax
            tgt = ring(p)
            y = mm(wfull[p])
            ymax = jnp.maximum(ymax, jnp.max(y))
            tile_buf[pl.ds(p, 1)] = y[None]
            r = pltpu.make_async_remote_copy(
                src_ref=tile_buf.at[p],
                dst_ref=out_ref.at[pl.ds(my * m_per, m_per), :],
                send_sem=a2a_send_sems.at[s_idx],
                recv_sem=a2a_recv_sems.at[s_idx],
                device_id=(tgt,),
                device_id_type=pl.DeviceIdType.MESH,
            )
            r.start()
            a2a_rdmas.append(r)

        kh = k // 2
        cw_d = [None] * CW
        ccw_d = [None] * CCW
        for s in range(CW + 1):
            if s >= 1:
                cw_d[s - 1].wait_recv()
                ccw_d[s - 1].wait_recv()
            if s < CW:
                slot = (my_r - s + N_DEV) % N_DEV
                src1 = (wfull.at[slot] if s < CW - 1
                        else wfull.at[slot, pl.ds(0, kh), :])
                cw_d[s] = pltpu.make_async_remote_copy(
                    src_ref=src1,
                    dst_ref=src1,
                    send_sem=cw_send.at[s],
                    recv_sem=cw_recv.at[s],
                    device_id=(right,),
                    device_id_type=pl.DeviceIdType.MESH,
                )
                cw_d[s].start()
                slot2 = (my_r + s) % N_DEV
                src2 = (wfull.at[slot2] if s < CCW - 1
                        else wfull.at[slot2, pl.ds(kh, kh), :])
                ccw_d[s] = pltpu.make_async_remote_copy(
                    src_ref=src2,
                    dst_ref=src2,
                    send_sem=ccw_send.at[s],
                    recv_sem=ccw_recv.at[s],
                    device_id=(left,),
                    device_id_type=pl.DeviceIdType.MESH,
                )
                ccw_d[s].start()
            if s == 0:
                y0 = mm(w_ref[...])
                ymax = jnp.maximum(ymax, jnp.max(y0))
                out_ref[pl.ds(my * m_per, m_per), :] = y0
            elif s < CW:
                send_tile((my_r - s + N_DEV) % N_DEV, 15 - s)
                send_tile((my_r + s) % N_DEV, s - 1)
            else:
                send_tile((my_r + CW) % N_DEV, CW - 1)
        for r in cw_d:
            r.wait_send()
        for r in ccw_d:
            r.wait_send()
        for r in a2a_rdmas:
            r.wait_send()

        amax_src[...] = jnp.full((8, 128), ymax, dtype=jnp.float32)
        amax_rdmas = []
        for o in range(1, N_DEV):
            tgt = ring((my_r + o) % N_DEV)
            r = pltpu.make_async_remote_copy(
                src_ref=amax_src,
                dst_ref=amax_buf.at[o - 1],
                send_sem=amax_send_sems.at[o - 1],
                recv_sem=amax_recv_sems.at[o - 1],
                device_id=(tgt,),
                device_id_type=pl.DeviceIdType.MESH,
            )
            r.start()
            amax_rdmas.append(r)
        for r in amax_rdmas:
            r.wait_send()
        for r in amax_rdmas:
            r.wait_recv()
        gmax = ymax
        for o in range(1, N_DEV):
            gmax = jnp.maximum(gmax, amax_buf[o - 1, 0, 0])

        for r in a2a_rdmas:
            r.wait_recv()

        scale = gmax / jnp.float32(127.0)
        y_all = jnp.maximum(out_ref[...], 0.0)
        q = jnp.clip(jnp.round(y_all / scale), -127.0, 127.0)
        out_ref[...] = q * scale

    return pl.pallas_call(
        body,
        out_shape=jax.ShapeDtypeStruct((N_DEV * m_per, n_per), jnp.float32),
        in_specs=[
            pl.BlockSpec(memory_space=pltpu.SMEM),
            pl.BlockSpec(memory_space=pltpu.SMEM),
            pl.BlockSpec(memory_space=pltpu.VMEM),
            pl.BlockSpec(memory_space=pltpu.VMEM),
        ],
        out_specs=pl.BlockSpec(memory_space=pltpu.VMEM),
        scratch_shapes=[
            pltpu.VMEM((N_DEV, k, n_per), jnp.float32),
            pltpu.VMEM((N_DEV, m_per, n_per), jnp.float32),
            pltpu.SemaphoreType.DMA((CW,)),
            pltpu.SemaphoreType.DMA((CW,)),
            pltpu.SemaphoreType.DMA((CCW,)),
            pltpu.SemaphoreType.DMA((CCW,)),
            pltpu.SemaphoreType.DMA((N_DEV - 1,)),
            pltpu.SemaphoreType.DMA((N_DEV - 1,)),
            pltpu.VMEM((8, 128), jnp.float32),
            pltpu.VMEM((N_DEV - 1, 8, 128), jnp.float32),
            pltpu.SemaphoreType.DMA((N_DEV - 1,)),
            pltpu.SemaphoreType.DMA((N_DEV - 1,)),
        ],
        compiler_params=pltpu.CompilerParams(
            collective_id=0, vmem_limit_bytes=56 * 1024 * 1024,
        ),
    )(jnp.array(RING, dtype=jnp.int32), jnp.array(POS, dtype=jnp.int32),
      x, w_mat)


# device time: 214179 ns/iter; 3.3774x vs baseline; 1.0265x over previous
import jax
import jax.numpy as jnp
from jax import lax
from jax.experimental import pallas as pl
from jax.experimental.pallas import tpu as pltpu

N_DEV = 16
CW = 8
CCW = 8

RING = [0, 1, 2, 6, 5, 4, 8, 9, 10, 14, 13, 12, 15, 11, 7, 3]
POS = [0] * N_DEV
for _r, _l in enumerate(RING):
    POS[_l] = _r


def kernel(x, w_mat):
    m_per, k = x.shape
    _, n_per = w_mat.shape

    def body(ring_ref, pos_ref, x_ref, w_ref, out_ref,
             wfull, tile_buf, tq_buf, q_recv,
             cw_send, cw_recv, ccw_send, ccw_recv,
             a2a_send_sems, a2a_recv_sems,
             amax_src, amax_buf, amax_send_sems, amax_recv_sems):
        my = lax.axis_index("i")

        def ring(i):
            return ring_ref[i]

        my_r = pos_ref[my]
        right = ring((my_r + 1) % N_DEV)
        left = ring((my_r - 1 + N_DEV) % N_DEV)

        barrier_sem = pltpu.get_barrier_semaphore()
        for nbr in [left, right]:
            pl.semaphore_signal(
                barrier_sem, inc=1,
                device_id=(nbr,), device_id_type=pl.DeviceIdType.MESH,
            )
        pl.semaphore_wait(barrier_sem, 2)

        def mm(a):
            return lax.dot_general(
                x_ref[...], a, (((1,), (0,)), ((), ())),
                precision=lax.Precision.HIGHEST,
                preferred_element_type=jnp.float32,
            )

        wfull[pl.ds(my_r, 1)] = w_ref[...][None]

        ymax = jnp.float32(0.0)
        tile_meta = []

        def send_tile(p, s_idx):
            nonlocal ymax
            y = mm(wfull[p])
            ymax = jnp.maximum(ymax, jnp.max(y))
            tile_buf[pl.ds(p, 1)] = y[None]
            tile_meta.append((p, s_idx))

        kh = k // 2
        cw_d = [None] * CW
        ccw_d = [None] * CCW
        for s in range(CW + 1):
            if s >= 1:
                cw_d[s - 1].wait_recv()
                ccw_d[s - 1].wait_recv()
            if s < CW:
                slot = (my_r - s + N_DEV) % N_DEV
                src1 = (wfull.at[slot] if s < CW - 1
                        else wfull.at[slot, pl.ds(0, kh), :])
                cw_d[s] = pltpu.make_async_remote_copy(
                    src_ref=src1,
                    dst_ref=src1,
                    send_sem=cw_send.at[s],
                    recv_sem=cw_recv.at[s],
                    device_id=(right,),
                    device_id_type=pl.DeviceIdType.MESH,
                )
                cw_d[s].start()
                slot2 = (my_r + s) % N_DEV
                src2 = (wfull.at[slot2] if s < CCW - 1
                        else wfull.at[slot2, pl.ds(kh, kh), :])
                ccw_d[s] = pltpu.make_async_remote_copy(
                    src_ref=src2,
                    dst_ref=src2,
                    send_sem=ccw_send.at[s],
                    recv_sem=ccw_recv.at[s],
                    device_id=(left,),
                    device_id_type=pl.DeviceIdType.MESH,
                )
                ccw_d[s].start()
            if s == 0:
                y0 = mm(w_ref[...])
                ymax = jnp.maximum(ymax, jnp.max(y0))
                out_ref[pl.ds(my * m_per, m_per), :] = y0
            elif s < CW:
                send_tile((my_r - s + N_DEV) % N_DEV, 15 - s)
                send_tile((my_r + s) % N_DEV, s - 1)
            else:
                send_tile((my_r + CW) % N_DEV, CW - 1)
        for r in cw_d:
            r.wait_send()
        for r in ccw_d:
            r.wait_send()

        amax_src[...] = jnp.full((8, 128), ymax, dtype=jnp.float32)
        amax_rdmas = []
        for o in range(1, N_DEV):
            tgt = ring((my_r + o) % N_DEV)
            r = pltpu.make_async_remote_copy(
                src_ref=amax_src,
                dst_ref=amax_buf.at[o - 1],
                send_sem=amax_send_sems.at[o - 1],
                recv_sem=amax_recv_sems.at[o - 1],
                device_id=(tgt,),
                device_id_type=pl.DeviceIdType.MESH,
            )
            r.start()
            amax_rdmas.append(r)
        for r in amax_rdmas:
            r.wait_send()
        for r in amax_rdmas:
            r.wait_recv()
        gmax = ymax
        for o in range(1, N_DEV):
            gmax = jnp.maximum(gmax, amax_buf[o - 1, 0, 0])
        scale = gmax / jnp.float32(127.0)

        def quant(y):
            return jnp.clip(
                jnp.round(jnp.maximum(y, 0.0) / scale), -127.0, 127.0
            )

        q_rdmas = []
        for p, s_idx in tile_meta:
            tq_buf[pl.ds(s_idx, 1)] = quant(tile_buf[p])[None].astype(jnp.int8)
            r = pltpu.make_async_remote_copy(
                src_ref=tq_buf.at[s_idx],
                dst_ref=q_recv.at[s_idx],
                send_sem=a2a_send_sems.at[s_idx],
                recv_sem=a2a_recv_sems.at[s_idx],
                device_id=(ring(p),),
                device_id_type=pl.DeviceIdType.MESH,
            )
            r.start()
            q_rdmas.append(r)

        own = out_ref[pl.ds(my * m_per, m_per), :]
        out_ref[pl.ds(my * m_per, m_per), :] = quant(own) * scale

        for r in q_rdmas:
            r.wait_send()
        for r in q_rdmas:
            r.wait_recv()
        for o in range(1, N_DEV):
            sender = ring((my_r - o + N_DEV) % N_DEV)
            out_ref[pl.ds(sender * m_per, m_per), :] = (
                q_recv[o - 1].astype(jnp.float32) * scale
            )

    return pl.pallas_call(
        body,
        out_shape=jax.ShapeDtypeStruct((N_DEV * m_per, n_per), jnp.float32),
        in_specs=[
            pl.BlockSpec(memory_space=pltpu.SMEM),
            pl.BlockSpec(memory_space=pltpu.SMEM),
            pl.BlockSpec(memory_space=pltpu.VMEM),
            pl.BlockSpec(memory_space=pltpu.VMEM),
        ],
        out_specs=pl.BlockSpec(memory_space=pltpu.VMEM),
        scratch_shapes=[
            pltpu.VMEM((N_DEV, k, n_per), jnp.float32),
            pltpu.VMEM((N_DEV, m_per, n_per), jnp.float32),
            pltpu.VMEM((N_DEV - 1, m_per, n_per), jnp.int8),
            pltpu.VMEM((N_DEV - 1, m_per, n_per), jnp.int8),
            pltpu.SemaphoreType.DMA((CW,)),
            pltpu.SemaphoreType.DMA((CW,)),
            pltpu.SemaphoreType.DMA((CCW,)),
            pltpu.SemaphoreType.DMA((CCW,)),
            pltpu.SemaphoreType.DMA((N_DEV - 1,)),
            pltpu.SemaphoreType.DMA((N_DEV - 1,)),
            pltpu.VMEM((8, 128), jnp.float32),
            pltpu.VMEM((N_DEV - 1, 8, 128), jnp.float32),
            pltpu.SemaphoreType.DMA((N_DEV - 1,)),
            pltpu.SemaphoreType.DMA((N_DEV - 1,)),
        ],
        compiler_params=pltpu.CompilerParams(
            collective_id=0, vmem_limit_bytes=56 * 1024 * 1024,
        ),
    )(jnp.array(RING, dtype=jnp.int32), jnp.array(POS, dtype=jnp.int32),
      x, w_mat)


# device time: 214142 ns/iter; 3.3780x vs baseline; 1.0002x over previous
import jax
import jax.numpy as jnp
from jax import lax
from jax.experimental import pallas as pl
from jax.experimental.pallas import tpu as pltpu

N_DEV = 16
CW = 8
CCW = 8

RING = [0, 1, 2, 6, 5, 4, 8, 9, 10, 14, 13, 12, 15, 11, 7, 3]
POS = [0] * N_DEV
for _r, _l in enumerate(RING):
    POS[_l] = _r


def kernel(x, w_mat):
    m_per, k = x.shape
    _, n_per = w_mat.shape

    def body(ring_ref, pos_ref, x_ref, w_ref, out_ref,
             wfull, tile_buf, tq_buf, q_recv,
             cw_send, cw_recv, ccw_send, ccw_recv,
             a2a_send_sems, a2a_recv_sems,
             amax_src, amax_buf, amax_send_sems, amax_recv_sems):
        my = lax.axis_index("i")

        def ring(i):
            return ring_ref[i]

        my_r = pos_ref[my]
        right = ring((my_r + 1) % N_DEV)
        left = ring((my_r - 1 + N_DEV) % N_DEV)

        barrier_sem = pltpu.get_barrier_semaphore()
        for nbr in [left, right]:
            pl.semaphore_signal(
                barrier_sem, inc=1,
                device_id=(nbr,), device_id_type=pl.DeviceIdType.MESH,
            )
        pl.semaphore_wait(barrier_sem, 2)

        def mm(a):
            return lax.dot_general(
                x_ref[...], a, (((1,), (0,)), ((), ())),
                precision=lax.Precision.HIGHEST,
                preferred_element_type=jnp.float32,
            )

        wfull[pl.ds(my_r, 1)] = w_ref[...][None]

        ymax = jnp.float32(0.0)
        tile_meta = []

        def send_tile(p, s_idx):
            nonlocal ymax
            y = jnp.maximum(mm(wfull[p]), 0.0)
            ymax = jnp.maximum(ymax, jnp.max(y))
            tile_buf[pl.ds(p, 1)] = y[None]
            tile_meta.append((p, s_idx))

        kh = k // 2
        cw_d = [None] * CW
        ccw_d = [None] * CCW
        for s in range(CW + 1):
            if s >= 1:
                cw_d[s - 1].wait_recv()
            if s < CW:
                slot = (my_r - s + N_DEV) % N_DEV
                src1 = (wfull.at[slot] if s < CW - 1
                        else wfull.at[slot, pl.ds(0, kh), :])
                cw_d[s] = pltpu.make_async_remote_copy(
                    src_ref=src1,
                    dst_ref=src1,
                    send_sem=cw_send.at[s],
                    recv_sem=cw_recv.at[s],
                    device_id=(right,),
                    device_id_type=pl.DeviceIdType.MESH,
                )
                cw_d[s].start()
            if s >= 1:
                ccw_d[s - 1].wait_recv()
            if s < CW:
                slot2 = (my_r + s) % N_DEV
                src2 = (wfull.at[slot2] if s < CCW - 1
                        else wfull.at[slot2, pl.ds(kh, kh), :])
                ccw_d[s] = pltpu.make_async_remote_copy(
                    src_ref=src2,
                    dst_ref=src2,
                    send_sem=ccw_send.at[s],
                    recv_sem=ccw_recv.at[s],
                    device_id=(left,),
                    device_id_type=pl.DeviceIdType.MESH,
                )
                ccw_d[s].start()
            if s == 0:
                y0 = jnp.maximum(mm(w_ref[...]), 0.0)
                ymax = jnp.maximum(ymax, jnp.max(y0))
                out_ref[pl.ds(my * m_per, m_per), :] = y0
            elif s < CW:
                send_tile((my_r - s + N_DEV) % N_DEV, 15 - s)
                send_tile((my_r + s) % N_DEV, s - 1)
            else:
                send_tile((my_r + CW) % N_DEV, CW - 1)
        for r in cw_d:
            r.wait_send()
        for r in ccw_d:
            r.wait_send()

        amax_src[...] = jnp.full((8, 128), ymax, dtype=jnp.float32)
        amax_rdmas = []
        for o in range(1, N_DEV):
            tgt = ring((my_r + o) % N_DEV)
            r = pltpu.make_async_remote_copy(
                src_ref=amax_src,
                dst_ref=amax_buf.at[o - 1],
                send_sem=amax_send_sems.at[o - 1],
                recv_sem=amax_recv_sems.at[o - 1],
                device_id=(tgt,),
                device_id_type=pl.DeviceIdType.MESH,
            )
            r.start()
            amax_rdmas.append(r)
        for r in amax_rdmas:
            r.wait_send()
        for r in amax_rdmas:
            r.wait_recv()
        gmax = ymax
        for o in range(1, N_DEV):
            gmax = jnp.maximum(gmax, amax_buf[o - 1, 0, 0])
        scale = gmax / jnp.float32(127.0)

        def quant(y):
            return jnp.clip(jnp.round(y / scale), 0.0, 127.0)

        q_rdmas = []
        for p, s_idx in tile_meta:
            tq_buf[pl.ds(s_idx, 1)] = quant(tile_buf[p])[None].astype(jnp.int8)
            r = pltpu.make_async_remote_copy(
                src_ref=tq_buf.at[s_idx],
                dst_ref=q_recv.at[s_idx],
                send_sem=a2a_send_sems.at[s_idx],
                recv_sem=a2a_recv_sems.at[s_idx],
                device_id=(ring(p),),
                device_id_type=pl.DeviceIdType.MESH,
            )
            r.start()
            q_rdmas.append(r)

        own = out_ref[pl.ds(my * m_per, m_per), :]
        out_ref[pl.ds(my * m_per, m_per), :] = quant(own) * scale

        for r in q_rdmas:
            r.wait_send()
        for r in q_rdmas:
            r.wait_recv()
        for o in range(1, N_DEV):
            sender = ring((my_r - o + N_DEV) % N_DEV)
            out_ref[pl.ds(sender * m_per, m_per), :] = (
                q_recv[o - 1].astype(jnp.float32) * scale
            )

    return pl.pallas_call(
        body,
        out_shape=jax.ShapeDtypeStruct((N_DEV * m_per, n_per), jnp.float32),
        in_specs=[
            pl.BlockSpec(memory_space=pltpu.SMEM),
            pl.BlockSpec(memory_space=pltpu.SMEM),
            pl.BlockSpec(memory_space=pltpu.VMEM),
            pl.BlockSpec(memory_space=pltpu.VMEM),
        ],
        out_specs=pl.BlockSpec(memory_space=pltpu.VMEM),
        scratch_shapes=[
            pltpu.VMEM((N_DEV, k, n_per), jnp.float32),
            pltpu.VMEM((N_DEV, m_per, n_per), jnp.float32),
            pltpu.VMEM((N_DEV - 1, m_per, n_per), jnp.int8),
            pltpu.VMEM((N_DEV - 1, m_per, n_per), jnp.int8),
            pltpu.SemaphoreType.DMA((CW,)),
            pltpu.SemaphoreType.DMA((CW,)),
            pltpu.SemaphoreType.DMA((CCW,)),
            pltpu.SemaphoreType.DMA((CCW,)),
            pltpu.SemaphoreType.DMA((N_DEV - 1,)),
            pltpu.SemaphoreType.DMA((N_DEV - 1,)),
            pltpu.VMEM((8, 128), jnp.float32),
            pltpu.VMEM((N_DEV - 1, 8, 128), jnp.float32),
            pltpu.SemaphoreType.DMA((N_DEV - 1,)),
            pltpu.SemaphoreType.DMA((N_DEV - 1,)),
        ],
        compiler_params=pltpu.CompilerParams(
            collective_id=0, vmem_limit_bytes=56 * 1024 * 1024,
        ),
    )(jnp.array(RING, dtype=jnp.int32), jnp.array(POS, dtype=jnp.int32),
      x, w_mat)


# device time: 209708 ns/iter; 3.4494x vs baseline; 1.0211x over previous
import jax
import jax.numpy as jnp
from jax import lax
from jax.experimental import pallas as pl
from jax.experimental.pallas import tpu as pltpu

N_DEV = 16
CW = 8
CCW = 8

RING = [0, 1, 2, 6, 5, 4, 8, 9, 10, 14, 13, 12, 15, 11, 7, 3]
POS = [0] * N_DEV
for _r, _l in enumerate(RING):
    POS[_l] = _r


def kernel(x, w_mat):
    m_per, k = x.shape
    _, n_per = w_mat.shape

    def body(ring_ref, pos_ref, x_ref, w_ref, out_ref,
             wfull, tile_buf, tq_buf, q_recv,
             cw_send, cw_recv, ccw_send, ccw_recv,
             a2a_send_sems, a2a_recv_sems,
             amax_src, amax_buf, amax_send_sems, amax_recv_sems):
        my = lax.axis_index("i")

        def ring(i):
            return ring_ref[i]

        my_r = pos_ref[my]
        right = ring((my_r + 1) % N_DEV)
        left = ring((my_r - 1 + N_DEV) % N_DEV)

        barrier_sem = pltpu.get_barrier_semaphore()
        for nbr in [left, right]:
            pl.semaphore_signal(
                barrier_sem, inc=1,
                device_id=(nbr,), device_id_type=pl.DeviceIdType.MESH,
            )
        pl.semaphore_wait(barrier_sem, 2)

        def mm(a):
            return lax.dot_general(
                x_ref[...], a, (((1,), (0,)), ((), ())),
                precision=lax.Precision.HIGHEST,
                preferred_element_type=jnp.float32,
            )

        wfull[pl.ds(my_r, 1)] = w_ref[...][None]

        ymax = jnp.float32(0.0)
        tile_meta = []

        def send_tile(p, s_idx):
            nonlocal ymax
            y = jnp.maximum(mm(wfull[p]), 0.0)
            ymax = jnp.maximum(ymax, jnp.max(y))
            tile_buf[pl.ds(p, 1)] = y[None]
            tile_meta.append((p, s_idx))

        kh = k // 2
        NC = 2 * CW - 1
        cw_d = [None] * NC
        ccw_d = [None] * NC
        for q in range(NC):
            d = q // 2
            if q >= 2:
                cw_d[q - 2].wait_recv()
            slot = (my_r - d + N_DEV) % N_DEV
            src1 = wfull.at[slot, pl.ds((q % 2) * kh, kh), :]
            cw_d[q] = pltpu.make_async_remote_copy(
                src_ref=src1,
                dst_ref=src1,
                send_sem=cw_send.at[q],
                recv_sem=cw_recv.at[q],
                device_id=(right,),
                device_id_type=pl.DeviceIdType.MESH,
            )
            cw_d[q].start()
            if q >= 2:
                ccw_d[q - 2].wait_recv()
            slot2 = (my_r + d) % N_DEV
            src2 = wfull.at[slot2, pl.ds((1 - q % 2) * kh, kh), :]
            ccw_d[q] = pltpu.make_async_remote_copy(
                src_ref=src2,
                dst_ref=src2,
                send_sem=ccw_send.at[q],
                recv_sem=ccw_recv.at[q],
                device_id=(left,),
                device_id_type=pl.DeviceIdType.MESH,
            )
            ccw_d[q].start()
            if q == 0:
                y0 = jnp.maximum(mm(w_ref[...]), 0.0)
                ymax = jnp.maximum(ymax, jnp.max(y0))
                out_ref[pl.ds(my * m_per, m_per), :] = y0
            elif q % 2 == 1 and q >= 3:
                dd = (q - 3) // 2
                send_tile((my_r - 1 - dd + N_DEV) % N_DEV, 14 - dd)
                send_tile((my_r + 1 + dd) % N_DEV, dd)
        for q in (NC - 2, NC - 1):
            cw_d[q].wait_recv()
            ccw_d[q].wait_recv()
        send_tile((my_r - 7 + N_DEV) % N_DEV, 8)
        send_tile((my_r + 7) % N_DEV, 6)
        send_tile((my_r + 8) % N_DEV, 7)
        for r in cw_d:
            r.wait_send()
        for r in ccw_d:
            r.wait_send()

        amax_src[...] = jnp.full((8, 128), ymax, dtype=jnp.float32)
        amax_rdmas = []
        for o in range(1, N_DEV):
            tgt = ring((my_r + o) % N_DEV)
            r = pltpu.make_async_remote_copy(
                src_ref=amax_src,
                dst_ref=amax_buf.at[o - 1],
                send_sem=amax_send_sems.at[o - 1],
                recv_sem=amax_recv_sems.at[o - 1],
                device_id=(tgt,),
                device_id_type=pl.DeviceIdType.MESH,
            )
            r.start()
            amax_rdmas.append(r)
        for r in amax_rdmas:
            r.wait_send()
        for r in amax_rdmas:
            r.wait_recv()
        gmax = ymax
        for o in range(1, N_DEV):
            gmax = jnp.maximum(gmax, amax_buf[o - 1, 0, 0])
        scale = gmax / jnp.float32(127.0)

        def quant(y):
            return jnp.clip(jnp.round(y / scale), 0.0, 127.0)

        q_rdmas = []
        for p, s_idx in tile_meta:
            tq_buf[pl.ds(s_idx, 1)] = quant(tile_buf[p])[None].astype(jnp.int8)
            r = pltpu.make_async_remote_copy(
                src_ref=tq_buf.at[s_idx],
                dst_ref=q_recv.at[s_idx],
                send_sem=a2a_send_sems.at[s_idx],
                recv_sem=a2a_recv_sems.at[s_idx],
                device_id=(ring(p),),
                device_id_type=pl.DeviceIdType.MESH,
            )
            r.start()
            q_rdmas.append(r)

        own = out_ref[pl.ds(my * m_per, m_per), :]
        out_ref[pl.ds(my * m_per, m_per), :] = quant(own) * scale

        for r in q_rdmas:
            r.wait_send()
        for r in q_rdmas:
            r.wait_recv()
        for o in range(1, N_DEV):
            sender = ring((my_r - o + N_DEV) % N_DEV)
            out_ref[pl.ds(sender * m_per, m_per), :] = (
                q_recv[o - 1].astype(jnp.float32) * scale
            )

    return pl.pallas_call(
        body,
        out_shape=jax.ShapeDtypeStruct((N_DEV * m_per, n_per), jnp.float32),
        in_specs=[
            pl.BlockSpec(memory_space=pltpu.SMEM),
            pl.BlockSpec(memory_space=pltpu.SMEM),
            pl.BlockSpec(memory_space=pltpu.VMEM),
            pl.BlockSpec(memory_space=pltpu.VMEM),
        ],
        out_specs=pl.BlockSpec(memory_space=pltpu.VMEM),
        scratch_shapes=[
            pltpu.VMEM((N_DEV, k, n_per), jnp.float32),
            pltpu.VMEM((N_DEV, m_per, n_per), jnp.float32),
            pltpu.VMEM((N_DEV - 1, m_per, n_per), jnp.int8),
            pltpu.VMEM((N_DEV - 1, m_per, n_per), jnp.int8),
            pltpu.SemaphoreType.DMA((2 * CW - 1,)),
            pltpu.SemaphoreType.DMA((2 * CW - 1,)),
            pltpu.SemaphoreType.DMA((2 * CCW - 1,)),
            pltpu.SemaphoreType.DMA((2 * CCW - 1,)),
            pltpu.SemaphoreType.DMA((N_DEV - 1,)),
            pltpu.SemaphoreType.DMA((N_DEV - 1,)),
            pltpu.VMEM((8, 128), jnp.float32),
            pltpu.VMEM((N_DEV - 1, 8, 128), jnp.float32),
            pltpu.SemaphoreType.DMA((N_DEV - 1,)),
            pltpu.SemaphoreType.DMA((N_DEV - 1,)),
        ],
        compiler_params=pltpu.CompilerParams(
            collective_id=0, vmem_limit_bytes=56 * 1024 * 1024,
        ),
    )(jnp.array(RING, dtype=jnp.int32), jnp.array(POS, dtype=jnp.int32),
      x, w_mat)
